# baseline (speedup 1.0000x reference)
import functools

import jax
import jax.numpy as jnp
from jax import lax
from jax.experimental import pallas as pl
from jax.experimental.pallas import tpu as pltpu
from jax.experimental.pallas import tpu_sc as plsc

EMB = 32
B = 4096
L = 200
WIDE = 128
RPT = WIDE // EMB

NC = 2
NS = 16
NW = NC * NS
SPW = B // NW
TOK = SPW * L
NFULL = L // 16
TAIL = L - NFULL * 16


def _sc_kernel(x_hbm, table_hbm, out_hbm, idx_v, sid_v, off_v, rows_v, out_v,
               sem_g1, sem_g2):
    c = lax.axis_index("c")
    s = lax.axis_index("s")
    wid = s * NC + c
    base_tok = wid * TOK

    pltpu.sync_copy(x_hbm.at[pl.ds(base_tok, TOK)], idx_v.at[pl.ds(0, TOK)])

    lane = lax.iota(jnp.int32, 16)

    def sentence(si, carry):
        sbase = si * L
        cnt = jnp.zeros((16,), jnp.int32)
        for k in range(NFULL + 1):
            v = idx_v[pl.ds(sbase + 16 * k, 16)]
            xm = jnp.where(v == 1, 0, v)
            if k == NFULL:
                valid = (xm != 0) & (lane < TAIL)
            else:
                valid = xm != 0
            cnt = cnt + plsc.all_reduce_population_count(valid)
            sid_v[pl.ds(16 * k, 16)] = xm >> 2
            off_v[pl.ds(16 * k, 16)] = (xm & 3) * EMB
        count = cnt.astype(jnp.float32)

        cp1 = pltpu.async_copy(table_hbm.at[sid_v.at[pl.ds(0, 128)]],
                               rows_v.at[pl.ds(0, 128)], sem_g1)
        cp2 = pltpu.async_copy(table_hbm.at[sid_v.at[pl.ds(128, 72)]],
                               rows_v.at[pl.ds(128, 72)], sem_g2)
        cp1.wait()
        cp2.wait()

        def red(o, accs):
            a0, a1 = accs
            offs = off_v[pl.ds(o * 8, 16)]
            for j in range(8):
                r = o * 8 + j
                off = offs[j]
                a0 = a0 + rows_v[r, pl.ds(off, 16)]
                a1 = a1 + rows_v[r, pl.ds(off + 16, 16)]
            return a0, a1

        acc0, acc1 = lax.fori_loop(
            0, L // 8, red,
            (jnp.zeros((16,), jnp.float32), jnp.zeros((16,), jnp.float32)))

        scale = jnp.where(count > 0.0, 1.0 / jnp.maximum(count, 1.0), 0.0)
        orow = si // RPT
        ocol = (si % RPT) * EMB
        out_v[orow, pl.ds(ocol, 16)] = acc0 * scale
        out_v[orow, pl.ds(ocol + 16, 16)] = acc1 * scale
        return carry

    lax.fori_loop(0, SPW, sentence, 0)

    pltpu.sync_copy(out_v, out_hbm.at[pl.ds(wid * (SPW // RPT), SPW // RPT)])


@jax.jit
def _run(x_flat, table_wide):
    mesh = plsc.VectorSubcoreMesh(core_axis_name="c", subcore_axis_name="s")
    kern = functools.partial(
        pl.kernel,
        out_type=jax.ShapeDtypeStruct((B // RPT, WIDE), jnp.float32),
        mesh=mesh,
        compiler_params=pltpu.CompilerParams(needs_layout_passes=False,
                                             use_tc_tiling_on_sc=True),
        scratch_types=[
            pltpu.VMEM((TOK + 16,), jnp.int32),
            pltpu.VMEM((208,), jnp.int32),
            pltpu.VMEM((208,), jnp.int32),
            pltpu.VMEM((L, WIDE), jnp.float32),
            pltpu.VMEM((SPW // RPT, WIDE), jnp.float32),
            pltpu.SemaphoreType.DMA,
            pltpu.SemaphoreType.DMA,
        ],
    )(_sc_kernel)
    return kern(x_flat, table_wide)


def kernel(x, table):
    out_wide = _run(x.reshape(-1), table.reshape(-1, WIDE))
    return out_wide.reshape(B, EMB)


# speedup vs baseline: 1.0999x; 1.0999x over previous
import functools

import jax
import jax.numpy as jnp
from jax import lax
from jax.experimental import pallas as pl
from jax.experimental.pallas import tpu as pltpu
from jax.experimental.pallas import tpu_sc as plsc

EMB = 32
B = 4096
L = 200

NC = 2
NS = 16
NW = NC * NS
SPW = B // NW
TOK = SPW * L
NFULL = L // 16
TAIL = L - NFULL * 16


def _sc_kernel(x_hbm, table_hbm, out_hbm, idx_v, sid_v, rows_v, out_v,
               sem_g1, sem_g2):
    c = lax.axis_index("c")
    s = lax.axis_index("s")
    wid = s * NC + c
    base_tok = wid * TOK

    pltpu.sync_copy(x_hbm.at[pl.ds(base_tok, TOK)], idx_v.at[pl.ds(0, TOK)])

    lane = lax.iota(jnp.int32, 16)

    def sentence(si, carry):
        sbase = si * L
        cnt = jnp.zeros((16,), jnp.int32)
        for k in range(NFULL + 1):
            v = idx_v[pl.ds(sbase + 16 * k, 16)]
            xm = jnp.where(v == 1, 0, v)
            if k == NFULL:
                valid = (xm != 0) & (lane < TAIL)
            else:
                valid = xm != 0
            cnt = cnt + plsc.all_reduce_population_count(valid)
            sid_v[pl.ds(16 * k, 16)] = xm
        count = cnt.astype(jnp.float32)

        cp1 = pltpu.async_copy(table_hbm.at[sid_v.at[pl.ds(0, 128)]],
                               rows_v.at[pl.ds(0, 128)], sem_g1)
        cp2 = pltpu.async_copy(table_hbm.at[sid_v.at[pl.ds(128, 72)]],
                               rows_v.at[pl.ds(128, 72)], sem_g2)
        cp1.wait()
        cp2.wait()

        def red(o, accs):
            ae, ao = accs
            for j in range(8):
                row = rows_v[o * 8 + j, :]
                e, od = plsc.unpack(row, format=plsc.PackFormat.INTERLEAVED)
                ae = ae + e
                ao = ao + od
            return ae, ao

        acc_e, acc_o = lax.fori_loop(
            0, L // 8, red,
            (jnp.zeros((16,), jnp.float32), jnp.zeros((16,), jnp.float32)))

        scale = jnp.where(count > 0.0, 1.0 / jnp.maximum(count, 1.0), 0.0)
        row_idx = jnp.full((16,), si, jnp.int32)
        plsc.store_scatter(out_v, [row_idx, 2 * lane], acc_e * scale)
        plsc.store_scatter(out_v, [row_idx, 2 * lane + 1], acc_o * scale)
        return carry

    lax.fori_loop(0, SPW, sentence, 0)

    pltpu.sync_copy(out_v, out_hbm.at[pl.ds(wid * SPW, SPW)])


@jax.jit
def _run(x_flat, table_bf):
    mesh = plsc.VectorSubcoreMesh(core_axis_name="c", subcore_axis_name="s")
    kern = functools.partial(
        pl.kernel,
        out_type=jax.ShapeDtypeStruct((B, EMB), jnp.float32),
        mesh=mesh,
        compiler_params=pltpu.CompilerParams(needs_layout_passes=False,
                                             use_tc_tiling_on_sc=False),
        scratch_types=[
            pltpu.VMEM((TOK + 16,), jnp.int32),
            pltpu.VMEM((208,), jnp.int32),
            pltpu.VMEM((L, EMB), jnp.bfloat16),
            pltpu.VMEM((SPW, EMB), jnp.float32),
            pltpu.SemaphoreType.DMA,
            pltpu.SemaphoreType.DMA,
        ],
    )(_sc_kernel)
    return kern(x_flat, table_bf)


def kernel(x, table):
    return _run(x.reshape(-1), table.astype(jnp.bfloat16))


# speedup vs baseline: 1.3885x; 1.2624x over previous
import functools

import jax
import jax.numpy as jnp
from jax import lax
from jax.experimental import pallas as pl
from jax.experimental.pallas import tpu as pltpu
from jax.experimental.pallas import tpu_sc as plsc

EMB = 32
B = 4096
L = 200

NC = 2
NS = 16
NW = NC * NS
SPW = B // NW
TOK = SPW * L
NFULL = L // 16
TAIL = L - NFULL * 16
G1 = 128
G2 = L - G1


def _sc_kernel(x_hbm, table_hbm, out_hbm, idx_v, sid_v, cnt_v, rows_v, out_v,
               sem0, sem1):
    c = lax.axis_index("c")
    s = lax.axis_index("s")
    wid = s * NC + c
    base_tok = wid * TOK

    pltpu.sync_copy(x_hbm.at[pl.ds(base_tok, TOK)], idx_v.at[pl.ds(0, TOK)])

    lane = lax.iota(jnp.int32, 16)
    sems = (sem0, sem1)

    def issue(si, slot):
        sbase = si * L
        cnt = jnp.zeros((16,), jnp.int32)
        for k in range(NFULL + 1):
            v = idx_v[pl.ds(sbase + 16 * k, 16)]
            xm = jnp.where(v == 1, 0, v)
            if k == NFULL:
                valid = (xm != 0) & (lane < TAIL)
            else:
                valid = xm != 0
            cnt = cnt + plsc.all_reduce_population_count(valid)
            sid_v[slot, pl.ds(16 * k, 16)] = xm
        cnt_v[slot, pl.ds(0, 16)] = cnt
        pltpu.async_copy(table_hbm.at[sid_v.at[slot, pl.ds(0, G1)]],
                         rows_v.at[slot, pl.ds(0, G1)], sems[slot])
        pltpu.async_copy(table_hbm.at[sid_v.at[slot, pl.ds(G1, G2)]],
                         rows_v.at[slot, pl.ds(G1, G2)], sems[slot])

    def drain(si, slot):
        pltpu.make_async_copy(table_hbm.at[sid_v.at[slot, pl.ds(0, G1)]],
                              rows_v.at[slot, pl.ds(0, G1)],
                              sems[slot]).wait()
        pltpu.make_async_copy(table_hbm.at[sid_v.at[slot, pl.ds(G1, G2)]],
                              rows_v.at[slot, pl.ds(G1, G2)],
                              sems[slot]).wait()

        def red(o, accs):
            a0, a1 = accs
            for j in range(8):
                r = o * 8 + j
                a0 = a0 + rows_v[slot, r, pl.ds(0, 16)]
                a1 = a1 + rows_v[slot, r, pl.ds(16, 16)]
            return a0, a1

        acc0, acc1 = lax.fori_loop(
            0, L // 8, red,
            (jnp.zeros((16,), jnp.float32), jnp.zeros((16,), jnp.float32)))

        count = cnt_v[slot, pl.ds(0, 16)].astype(jnp.float32)
        scale = jnp.where(count > 0.0, 1.0 / jnp.maximum(count, 1.0), 0.0)
        out_v[si, pl.ds(0, 16)] = acc0 * scale
        out_v[si, pl.ds(16, 16)] = acc1 * scale

    issue(0, 0)

    def body(k, carry):
        s0 = 2 * k
        issue(s0 + 1, 1)
        drain(s0, 0)

        @pl.when(s0 + 2 < SPW)
        def _():
            issue(s0 + 2, 0)

        drain(s0 + 1, 1)
        return carry

    lax.fori_loop(0, SPW // 2, body, 0)

    pltpu.sync_copy(out_v, out_hbm.at[pl.ds(wid * SPW, SPW)])


@jax.jit
def _run(x_flat, table):
    mesh = plsc.VectorSubcoreMesh(core_axis_name="c", subcore_axis_name="s")
    kern = functools.partial(
        pl.kernel,
        out_type=jax.ShapeDtypeStruct((B, EMB), jnp.float32),
        mesh=mesh,
        compiler_params=pltpu.CompilerParams(needs_layout_passes=False,
                                             use_tc_tiling_on_sc=False),
        scratch_types=[
            pltpu.VMEM((TOK + 16,), jnp.int32),
            pltpu.VMEM((2, 208), jnp.int32),
            pltpu.VMEM((2, 16), jnp.int32),
            pltpu.VMEM((2, L, EMB), jnp.float32),
            pltpu.VMEM((SPW, EMB), jnp.float32),
            pltpu.SemaphoreType.DMA,
            pltpu.SemaphoreType.DMA,
        ],
    )(_sc_kernel)
    return kern(x_flat, table)


def kernel(x, table):
    return _run(x.reshape(-1), table)
